# Initial kernel scaffold; baseline (speedup 1.0000x reference)
import jax
import jax.numpy as jnp
from jax.experimental import pallas as pl


def kernel(contexts, indices, leaf_table, path_table, W_fc, a, W_out, b_out):
    raise NotImplementedError("write your pallas kernel here")


# speedup vs baseline: 3.3415x; 3.3415x over previous
import functools

import jax
import jax.numpy as jnp
from jax import lax
from jax.experimental import pallas as pl
from jax.experimental.pallas import tpu as pltpu
from jax.experimental.pallas import tpu_sc as plsc

NUM_SEG = 10000
SEG_BLOCK = 128
ROW_CHUNK = 512
MLP_BLOCK = 512
SC_CHUNK = 80
NUM_WORKERS = 32


def _sc_gather(c0, c1, c2, leaf_table, path_table):
    n = c0.shape[0]
    d = leaf_table.shape[1]
    per_w = n // NUM_WORKERS
    n_chunks = per_w // SC_CHUNK
    mesh = plsc.VectorSubcoreMesh(core_axis_name="c", subcore_axis_name="s")
    row_t = jax.ShapeDtypeStruct((n, d), jnp.float32)

    @functools.partial(
        pl.kernel,
        mesh=mesh,
        out_type=(row_t, row_t, row_t),
        scratch_types=[
            pltpu.VMEM((SC_CHUNK,), jnp.int32),
            pltpu.VMEM((SC_CHUNK,), jnp.int32),
            pltpu.VMEM((SC_CHUNK,), jnp.int32),
            pltpu.VMEM((SC_CHUNK, d), jnp.float32),
            pltpu.VMEM((SC_CHUNK, d), jnp.float32),
            pltpu.VMEM((SC_CHUNK, d), jnp.float32),
            pltpu.SemaphoreType.DMA,
            pltpu.SemaphoreType.DMA,
            pltpu.SemaphoreType.DMA,
        ],
    )
    def gather_kernel(c0_h, c1_h, c2_h, leaf_h, path_h, o0_h, o1_h, o2_h,
                      i0, i1, i2, r0, r1, r2, s0, s1, s2):
        wid = lax.axis_index("s") * 2 + lax.axis_index("c")
        base = wid * per_w

        def body(c, carry):
            off = base + c * SC_CHUNK
            pltpu.sync_copy(c0_h.at[pl.ds(off, SC_CHUNK)], i0)
            pltpu.sync_copy(c1_h.at[pl.ds(off, SC_CHUNK)], i1)
            pltpu.sync_copy(c2_h.at[pl.ds(off, SC_CHUNK)], i2)
            cp0 = pltpu.async_copy(leaf_h.at[i0], r0, s0)
            cp1 = pltpu.async_copy(path_h.at[i1], r1, s1)
            cp2 = pltpu.async_copy(leaf_h.at[i2], r2, s2)
            cp0.wait()
            cp1.wait()
            cp2.wait()
            pltpu.sync_copy(r0, o0_h.at[pl.ds(off, SC_CHUNK)])
            pltpu.sync_copy(r1, o1_h.at[pl.ds(off, SC_CHUNK)])
            pltpu.sync_copy(r2, o2_h.at[pl.ds(off, SC_CHUNK)])
            return carry

        lax.fori_loop(0, n_chunks, body, 0)

    return gather_kernel(c0, c1, c2, leaf_table, path_table)


def _mlp(ll, pm, lr, w1t, w2t, w3t, a2):
    n, d = ll.shape
    code = w1t.shape[1]
    grid = (n // MLP_BLOCK,)

    def body(ll_r, pm_r, lr_r, w1_r, w2_r, w3_r, a_r, h_r, s_r):
        z = jnp.dot(ll_r[...], w1_r[...], preferred_element_type=jnp.float32)
        z = z + jnp.dot(pm_r[...], w2_r[...], preferred_element_type=jnp.float32)
        z = z + jnp.dot(lr_r[...], w3_r[...], preferred_element_type=jnp.float32)
        h = jnp.tanh(z)
        h_r[...] = h
        s_r[...] = lax.dot_general(
            a_r[...], h, (((1,), (1,)), ((), ())),
            preferred_element_type=jnp.float32)

    return pl.pallas_call(
        body,
        grid=grid,
        in_specs=[
            pl.BlockSpec((MLP_BLOCK, d), lambda i: (i, 0)),
            pl.BlockSpec((MLP_BLOCK, d), lambda i: (i, 0)),
            pl.BlockSpec((MLP_BLOCK, d), lambda i: (i, 0)),
            pl.BlockSpec((d, code), lambda i: (0, 0)),
            pl.BlockSpec((d, code), lambda i: (0, 0)),
            pl.BlockSpec((d, code), lambda i: (0, 0)),
            pl.BlockSpec((1, code), lambda i: (0, 0)),
        ],
        out_specs=[
            pl.BlockSpec((MLP_BLOCK, code), lambda i: (i, 0)),
            pl.BlockSpec((1, MLP_BLOCK), lambda i: (0, i)),
        ],
        out_shape=[
            jax.ShapeDtypeStruct((n, code), jnp.float32),
            jax.ShapeDtypeStruct((1, n), jnp.float32),
        ],
    )(ll, pm, lr, w1t, w2t, w3t, a2)


def _segment_out(h, score, idx2, bounds, w_out, b_out2, num_blocks):
    n, code = h.shape
    out_dim = w_out.shape[0]
    seg_pad = num_blocks * SEG_BLOCK

    def body(bounds_r, h_r, s_r, i_r, wout_r, bout_r, out_r,
             hbuf, sbuf, ibuf, sem_h, sem_s, sem_i):
        b = pl.program_id(0)
        r0 = bounds_r[b]
        r1 = bounds_r[b + 1]
        c_lo = r0 // ROW_CHUNK
        c_hi = lax.div(r1 + ROW_CHUNK - 1, ROW_CHUNK)
        seg0 = b * SEG_BLOCK

        def chunk(c, carry):
            m, dnm, acc = carry
            off = c * ROW_CHUNK
            cph = pltpu.make_async_copy(h_r.at[pl.ds(off, ROW_CHUNK)], hbuf, sem_h)
            cps = pltpu.make_async_copy(s_r.at[:, pl.ds(off, ROW_CHUNK)], sbuf, sem_s)
            cpi = pltpu.make_async_copy(i_r.at[:, pl.ds(off, ROW_CHUNK)], ibuf, sem_i)
            cph.start()
            cps.start()
            cpi.start()
            cph.wait()
            cps.wait()
            cpi.wait()
            rel = ibuf[...] - seg0
            sc = sbuf[...]
            rows = lax.broadcasted_iota(jnp.int32, (SEG_BLOCK, ROW_CHUNK), 0)
            onehot = rel == rows
            mc = jnp.max(jnp.where(onehot, sc, -1e30), axis=1, keepdims=True)
            m_new = jnp.maximum(m, mc)
            alpha = jnp.exp(m - m_new)
            ex = jnp.exp(jnp.where(onehot, sc - m_new, -1e30))
            dnm = dnm * alpha + jnp.sum(ex, axis=1, keepdims=True)
            acc = acc * alpha + jnp.dot(
                ex, hbuf[...], preferred_element_type=jnp.float32)
            return m_new, dnm, acc

        m0 = jnp.full((SEG_BLOCK, 1), -1e30, jnp.float32)
        d0 = jnp.zeros((SEG_BLOCK, 1), jnp.float32)
        a0 = jnp.zeros((SEG_BLOCK, code), jnp.float32)
        m, dnm, acc = lax.fori_loop(c_lo, c_hi, chunk, (m0, d0, a0))
        v = jnp.where(dnm > 0, acc / jnp.where(dnm > 0, dnm, 1.0), 0.0)
        out = lax.dot_general(
            v, wout_r[...], (((1,), (1,)), ((), ())),
            preferred_element_type=jnp.float32)
        out_r[...] = out + bout_r[...]

    return pl.pallas_call(
        body,
        grid=(num_blocks,),
        in_specs=[
            pl.BlockSpec(memory_space=pltpu.MemorySpace.SMEM),
            pl.BlockSpec(memory_space=pltpu.MemorySpace.HBM),
            pl.BlockSpec(memory_space=pltpu.MemorySpace.HBM),
            pl.BlockSpec(memory_space=pltpu.MemorySpace.HBM),
            pl.BlockSpec((out_dim, code), lambda b: (0, 0)),
            pl.BlockSpec((1, out_dim), lambda b: (0, 0)),
        ],
        out_specs=pl.BlockSpec((SEG_BLOCK, out_dim), lambda b: (b, 0)),
        out_shape=jax.ShapeDtypeStruct((seg_pad, out_dim), jnp.float32),
        scratch_shapes=[
            pltpu.VMEM((ROW_CHUNK, code), jnp.float32),
            pltpu.VMEM((1, ROW_CHUNK), jnp.float32),
            pltpu.VMEM((1, ROW_CHUNK), jnp.int32),
            pltpu.SemaphoreType.DMA,
            pltpu.SemaphoreType.DMA,
            pltpu.SemaphoreType.DMA,
        ],
    )(bounds, h, score, idx2, w_out, b_out2)


def kernel(contexts, indices, leaf_table, path_table, W_fc, a, W_out, b_out):
    n = contexts.shape[0]
    d = leaf_table.shape[1]
    num_blocks = (NUM_SEG + SEG_BLOCK - 1) // SEG_BLOCK

    c0 = contexts[:, 0]
    c1 = contexts[:, 1]
    c2 = contexts[:, 2]
    ll, pm, lr = _sc_gather(c0, c1, c2, leaf_table, path_table)

    wt = W_fc.T
    h, score = _mlp(ll, pm, lr, wt[:d], wt[d:2 * d], wt[2 * d:], a[None, :])

    seg_starts = jnp.arange(num_blocks, dtype=jnp.int32) * SEG_BLOCK
    bounds = jnp.concatenate([
        jnp.searchsorted(indices, seg_starts).astype(jnp.int32),
        jnp.array([n], jnp.int32),
    ])
    out_full = _segment_out(h, score, indices[None, :], bounds,
                            W_out, b_out[None, :], num_blocks)
    return out_full[:NUM_SEG]


# speedup vs baseline: 4.9102x; 1.4695x over previous
import functools

import jax
import jax.numpy as jnp
from jax import lax
from jax.experimental import pallas as pl
from jax.experimental.pallas import tpu as pltpu
from jax.experimental.pallas import tpu_sc as plsc

NUM_SEG = 10000
SEG_BLOCK = 128
ROW_CHUNK = 512
SC_CHUNK = 80
NUM_WORKERS = 32


def _sc_gather(c0, c1, c2, leaf_table, path_table):
    n = c0.shape[0]
    d = leaf_table.shape[1]
    per_w = n // NUM_WORKERS
    n_chunks = per_w // SC_CHUNK
    mesh = plsc.VectorSubcoreMesh(core_axis_name="c", subcore_axis_name="s")
    row_t = jax.ShapeDtypeStruct((n, d), jnp.float32)

    @functools.partial(
        pl.kernel,
        mesh=mesh,
        out_type=(row_t, row_t, row_t),
        scratch_types=[
            pltpu.VMEM((SC_CHUNK,), jnp.int32),
            pltpu.VMEM((SC_CHUNK,), jnp.int32),
            pltpu.VMEM((SC_CHUNK,), jnp.int32),
            pltpu.VMEM((SC_CHUNK, d), jnp.float32),
            pltpu.VMEM((SC_CHUNK, d), jnp.float32),
            pltpu.VMEM((SC_CHUNK, d), jnp.float32),
            pltpu.SemaphoreType.DMA,
            pltpu.SemaphoreType.DMA,
            pltpu.SemaphoreType.DMA,
        ],
    )
    def gather_kernel(c0_h, c1_h, c2_h, leaf_h, path_h, o0_h, o1_h, o2_h,
                      i0, i1, i2, r0, r1, r2, s0, s1, s2):
        wid = lax.axis_index("s") * 2 + lax.axis_index("c")
        base = wid * per_w

        def body(c, carry):
            off = base + c * SC_CHUNK
            pltpu.sync_copy(c0_h.at[pl.ds(off, SC_CHUNK)], i0)
            pltpu.sync_copy(c1_h.at[pl.ds(off, SC_CHUNK)], i1)
            pltpu.sync_copy(c2_h.at[pl.ds(off, SC_CHUNK)], i2)
            cp0 = pltpu.async_copy(leaf_h.at[i0], r0, s0)
            cp1 = pltpu.async_copy(path_h.at[i1], r1, s1)
            cp2 = pltpu.async_copy(leaf_h.at[i2], r2, s2)
            cp0.wait()
            cp1.wait()
            cp2.wait()
            pltpu.sync_copy(r0, o0_h.at[pl.ds(off, SC_CHUNK)])
            pltpu.sync_copy(r1, o1_h.at[pl.ds(off, SC_CHUNK)])
            pltpu.sync_copy(r2, o2_h.at[pl.ds(off, SC_CHUNK)])
            return carry

        lax.fori_loop(0, n_chunks, body, 0)

    return gather_kernel(c0, c1, c2, leaf_table, path_table)


def _segment_fused(ll, pm, lr, idx2, bounds, w1t, w2t, w3t, a2, w_out,
                   b_out2, num_blocks):
    n, d = ll.shape
    code = w1t.shape[1]
    out_dim = w_out.shape[0]
    seg_pad = num_blocks * SEG_BLOCK
    C = ROW_CHUNK

    def body(bounds_r, ll_r, pm_r, lr_r, i_r, w1_r, w2_r, w3_r, a_r,
             wout_r, bout_r, out_r,
             lbuf, pbuf, rbuf, ibuf, sem_l, sem_p, sem_r, sem_i):
        b = pl.program_id(0)
        r0 = bounds_r[b]
        r1 = bounds_r[b + 1]
        c_lo = r0 // C
        c_hi = lax.div(r1 + C - 1, C)
        seg0 = b * SEG_BLOCK

        def start(c, slot):
            off = c * C
            pltpu.make_async_copy(
                ll_r.at[pl.ds(off, C)], lbuf.at[slot], sem_l.at[slot]).start()
            pltpu.make_async_copy(
                pm_r.at[pl.ds(off, C)], pbuf.at[slot], sem_p.at[slot]).start()
            pltpu.make_async_copy(
                lr_r.at[pl.ds(off, C)], rbuf.at[slot], sem_r.at[slot]).start()
            pltpu.make_async_copy(
                i_r.at[:, pl.ds(off, C)], ibuf.at[slot], sem_i.at[slot]).start()

        def wait(c, slot):
            off = c * C
            pltpu.make_async_copy(
                ll_r.at[pl.ds(off, C)], lbuf.at[slot], sem_l.at[slot]).wait()
            pltpu.make_async_copy(
                pm_r.at[pl.ds(off, C)], pbuf.at[slot], sem_p.at[slot]).wait()
            pltpu.make_async_copy(
                lr_r.at[pl.ds(off, C)], rbuf.at[slot], sem_r.at[slot]).wait()
            pltpu.make_async_copy(
                i_r.at[:, pl.ds(off, C)], ibuf.at[slot], sem_i.at[slot]).wait()

        @pl.when(c_lo < c_hi)
        def _():
            start(c_lo, 0)

        def chunk(c, carry):
            m, dnm, acc = carry
            slot = lax.rem(c - c_lo, 2)

            @pl.when(c + 1 < c_hi)
            def _():
                start(c + 1, 1 - slot)

            wait(c, slot)
            z = jnp.dot(lbuf[slot], w1_r[...],
                        preferred_element_type=jnp.float32)
            z = z + jnp.dot(pbuf[slot], w2_r[...],
                            preferred_element_type=jnp.float32)
            z = z + jnp.dot(rbuf[slot], w3_r[...],
                            preferred_element_type=jnp.float32)
            h = jnp.tanh(z)
            sc = lax.dot_general(a_r[...], h, (((1,), (1,)), ((), ())),
                                 preferred_element_type=jnp.float32)
            rel = ibuf[slot] - seg0
            rows = lax.broadcasted_iota(jnp.int32, (SEG_BLOCK, C), 0)
            onehot = rel == rows
            mc = jnp.max(jnp.where(onehot, sc, -1e30), axis=1, keepdims=True)
            m_new = jnp.maximum(m, mc)
            alpha = jnp.exp(m - m_new)
            ex = jnp.exp(jnp.where(onehot, sc - m_new, -1e30))
            dnm = dnm * alpha + jnp.sum(ex, axis=1, keepdims=True)
            acc = acc * alpha + jnp.dot(
                ex, h, preferred_element_type=jnp.float32)
            return m_new, dnm, acc

        m0 = jnp.full((SEG_BLOCK, 1), -1e30, jnp.float32)
        d0 = jnp.zeros((SEG_BLOCK, 1), jnp.float32)
        a0 = jnp.zeros((SEG_BLOCK, code), jnp.float32)
        m, dnm, acc = lax.fori_loop(c_lo, c_hi, chunk, (m0, d0, a0))
        v = jnp.where(dnm > 0, acc / jnp.where(dnm > 0, dnm, 1.0), 0.0)
        out = lax.dot_general(
            v, wout_r[...], (((1,), (1,)), ((), ())),
            preferred_element_type=jnp.float32)
        out_r[...] = out + bout_r[...]

    return pl.pallas_call(
        body,
        grid=(num_blocks,),
        in_specs=[
            pl.BlockSpec(memory_space=pltpu.MemorySpace.SMEM),
            pl.BlockSpec(memory_space=pltpu.MemorySpace.HBM),
            pl.BlockSpec(memory_space=pltpu.MemorySpace.HBM),
            pl.BlockSpec(memory_space=pltpu.MemorySpace.HBM),
            pl.BlockSpec(memory_space=pltpu.MemorySpace.HBM),
            pl.BlockSpec((d, code), lambda b: (0, 0)),
            pl.BlockSpec((d, code), lambda b: (0, 0)),
            pl.BlockSpec((d, code), lambda b: (0, 0)),
            pl.BlockSpec((1, code), lambda b: (0, 0)),
            pl.BlockSpec((out_dim, code), lambda b: (0, 0)),
            pl.BlockSpec((1, out_dim), lambda b: (0, 0)),
        ],
        out_specs=pl.BlockSpec((SEG_BLOCK, out_dim), lambda b: (b, 0)),
        out_shape=jax.ShapeDtypeStruct((seg_pad, out_dim), jnp.float32),
        scratch_shapes=[
            pltpu.VMEM((2, C, d), jnp.float32),
            pltpu.VMEM((2, C, d), jnp.float32),
            pltpu.VMEM((2, C, d), jnp.float32),
            pltpu.VMEM((2, 1, C), jnp.int32),
            pltpu.SemaphoreType.DMA((2,)),
            pltpu.SemaphoreType.DMA((2,)),
            pltpu.SemaphoreType.DMA((2,)),
            pltpu.SemaphoreType.DMA((2,)),
        ],
    )(bounds, ll, pm, lr, idx2, w1t, w2t, w3t, a2, w_out, b_out2)


def kernel(contexts, indices, leaf_table, path_table, W_fc, a, W_out, b_out):
    n = contexts.shape[0]
    d = leaf_table.shape[1]
    num_blocks = (NUM_SEG + SEG_BLOCK - 1) // SEG_BLOCK

    c0 = contexts[:, 0]
    c1 = contexts[:, 1]
    c2 = contexts[:, 2]
    ll, pm, lr = _sc_gather(c0, c1, c2, leaf_table, path_table)

    wt = W_fc.T
    seg_starts = jnp.arange(num_blocks, dtype=jnp.int32) * SEG_BLOCK
    bounds = jnp.concatenate([
        jnp.searchsorted(indices, seg_starts).astype(jnp.int32),
        jnp.array([n], jnp.int32),
    ])
    out_full = _segment_fused(ll, pm, lr, indices[None, :], bounds,
                              wt[:d], wt[d:2 * d], wt[2 * d:], a[None, :],
                              W_out, b_out[None, :], num_blocks)
    return out_full[:NUM_SEG]
